# baseline (speedup 1.0000x reference)
import functools

import jax
import jax.numpy as jnp
from jax import lax
from jax.experimental import pallas as pl
from jax.experimental.pallas import tpu as pltpu
from jax.experimental.pallas import tpu_sc as plsc

NUM_ELEMENTS = 100
EMBED_DIM = 128

_B = 16384 * 200
_NC = 2
_NS = 16
_NW = _NC * _NS
_BPW = _B // _NW
_K = 8
_GROUP = _K * 128
_SUB = 256
_NSUB = _GROUP // _SUB
_NGROUP = _BPW // _GROUP
_IDX_ROWS_PER_W = _BPW // 128


def _make_sc_kernel():
    mesh = plsc.VectorSubcoreMesh(core_axis_name="c", subcore_axis_name="s")

    @functools.partial(
        pl.kernel,
        mesh=mesh,
        out_type=jax.ShapeDtypeStruct((_B, EMBED_DIM), jnp.float32),
        scratch_types=[
            pltpu.VMEM((_K, 128), jnp.int32),
            pltpu.VMEM((_SUB, EMBED_DIM), jnp.float32),
            pltpu.VMEM((_SUB, EMBED_DIM), jnp.float32),
            pltpu.SemaphoreType.DMA,
            pltpu.SemaphoreType.DMA,
            pltpu.SemaphoreType.DMA,
        ],
    )
    def emb(table_hbm, idx_hbm, out_hbm, idx_v, rows0, rows1, sem_g,
            sem_o0, sem_o1):
        wid = lax.axis_index("s") * _NC + lax.axis_index("c")
        idx_row_base = wid * _IDX_ROWS_PER_W
        out_base = wid * _BPW

        def drain_one(rows, sem_o):
            pltpu.make_async_copy(
                rows, out_hbm.at[pl.ds(out_base, _SUB)], sem_o
            ).wait()

        def subchunk(g, s, rows, sem_o, drain):
            if drain:
                drain_one(rows, sem_o)
            descs = [
                pltpu.async_copy(
                    table_hbm.at[idx_v.at[2 * s + j]],
                    rows.at[pl.ds(j * 128, 128)],
                    sem_g,
                )
                for j in range(2)
            ]
            for d in descs:
                d.wait()
            pltpu.async_copy(
                rows,
                out_hbm.at[pl.ds(out_base + g * _GROUP + s * _SUB, _SUB)],
                sem_o,
            )

        def group(g, first):
            pltpu.sync_copy(idx_hbm.at[pl.ds(idx_row_base + g * _K, _K)], idx_v)
            for s in range(_NSUB):
                rows, sem_o = (rows0, sem_o0) if s % 2 == 0 else (rows1, sem_o1)
                subchunk(g, s, rows, sem_o, drain=(not first) or s >= 2)

        group(0, True)

        def loop_body(g, carry):
            group(g, False)
            return carry

        lax.fori_loop(1, _NGROUP, loop_body, 0)

        drain_one(rows0, sem_o0)
        drain_one(rows1, sem_o1)

    return emb


_emb_kernel = _make_sc_kernel()


@jax.jit
def kernel(atom_type_array, embedding_table):
    idx2d = atom_type_array.astype(jnp.int32).reshape(_B // 128, 128)
    out = _emb_kernel(embedding_table, idx2d)
    return out.reshape(atom_type_array.shape + (EMBED_DIM,))


# speedup vs baseline: 4.8942x; 4.8942x over previous
import functools

import jax
import jax.numpy as jnp
from jax import lax
from jax.experimental import pallas as pl
from jax.experimental.pallas import tpu as pltpu
from jax.experimental.pallas import tpu_sc as plsc

NUM_ELEMENTS = 100
EMBED_DIM = 128

_B = 16384 * 200
_NC = 2
_NS = 16
_NW = _NC * _NS
_BPW = _B // _NW
_K = 8
_GROUP = _K * 128
_SUB = 256
_NSUB = _GROUP // _SUB
_NGROUP = _BPW // _GROUP
_IDX_ROWS_PER_W = _BPW // 128


def _make_sc_kernel():
    mesh = plsc.VectorSubcoreMesh(core_axis_name="c", subcore_axis_name="s")

    @functools.partial(
        pl.kernel,
        mesh=mesh,
        out_type=jax.ShapeDtypeStruct((_B, EMBED_DIM), jnp.float32),
        scratch_types=[
            pltpu.VMEM((_K, 128), jnp.int32),
            pltpu.VMEM((_SUB, EMBED_DIM), jnp.float32),
            pltpu.VMEM((_SUB, EMBED_DIM), jnp.float32),
            pltpu.VMEM_SHARED((NUM_ELEMENTS, EMBED_DIM), jnp.float32),
            pltpu.SemaphoreType.DMA,
            pltpu.SemaphoreType.DMA,
            pltpu.SemaphoreType.DMA,
        ],
    )
    def emb(table_hbm, idx_hbm, out_hbm, idx_v, rows0, rows1, table_sp,
            sem_g, sem_o0, sem_o1):
        sid = lax.axis_index("s")
        wid = sid * _NC + lax.axis_index("c")
        idx_row_base = wid * _IDX_ROWS_PER_W
        out_base = wid * _BPW

        @pl.when(sid == 0)
        def _():
            pltpu.sync_copy(table_hbm, table_sp)

        plsc.subcore_barrier()

        def drain_one(rows, sem_o):
            pltpu.make_async_copy(
                rows, out_hbm.at[pl.ds(out_base, _SUB)], sem_o
            ).wait()

        def subchunk(g, s, rows, sem_o, drain):
            if drain:
                drain_one(rows, sem_o)
            descs = [
                pltpu.async_copy(
                    table_sp.at[idx_v.at[2 * s + j]],
                    rows.at[pl.ds(j * 128, 128)],
                    sem_g,
                )
                for j in range(2)
            ]
            for d in descs:
                d.wait()
            pltpu.async_copy(
                rows,
                out_hbm.at[pl.ds(out_base + g * _GROUP + s * _SUB, _SUB)],
                sem_o,
            )

        def group(g, first):
            pltpu.sync_copy(idx_hbm.at[pl.ds(idx_row_base + g * _K, _K)], idx_v)
            for s in range(_NSUB):
                rows, sem_o = (rows0, sem_o0) if s % 2 == 0 else (rows1, sem_o1)
                subchunk(g, s, rows, sem_o, drain=(not first) or s >= 2)

        group(0, True)

        def loop_body(g, carry):
            group(g, False)
            return carry

        lax.fori_loop(1, _NGROUP, loop_body, 0)

        drain_one(rows0, sem_o0)
        drain_one(rows1, sem_o1)

    return emb


_emb_kernel = _make_sc_kernel()


@jax.jit
def kernel(atom_type_array, embedding_table):
    idx2d = atom_type_array.astype(jnp.int32).reshape(_B // 128, 128)
    out = _emb_kernel(embedding_table, idx2d)
    return out.reshape(atom_type_array.shape + (EMBED_DIM,))


# speedup vs baseline: 5.3064x; 1.0842x over previous
import functools

import jax
import jax.numpy as jnp
from jax import lax
from jax.experimental import pallas as pl
from jax.experimental.pallas import tpu as pltpu
from jax.experimental.pallas import tpu_sc as plsc

NUM_ELEMENTS = 100
EMBED_DIM = 128

_B = 16384 * 200
_NC = 2
_NS = 16
_NW = _NC * _NS
_BPW = _B // _NW
_K = 8
_SUB = 128
_NBUF = 4
_NGROUP = _BPW // (_K * _SUB)
_IDX_ROWS_PER_W = _BPW // 128


def _make_sc_kernel():
    mesh = plsc.VectorSubcoreMesh(core_axis_name="c", subcore_axis_name="s")

    @functools.partial(
        pl.kernel,
        mesh=mesh,
        out_type=jax.ShapeDtypeStruct((_B, EMBED_DIM), jnp.float32),
        scratch_types=(
            [pltpu.VMEM((_K, 128), jnp.int32)] * 2
            + [pltpu.VMEM((_SUB, EMBED_DIM), jnp.float32)] * _NBUF
            + [pltpu.VMEM_SHARED((NUM_ELEMENTS, EMBED_DIM), jnp.float32)]
            + [pltpu.SemaphoreType.DMA] * (2 + 2 * _NBUF)
        ),
    )
    def emb(table_hbm, idx_hbm, out_hbm, idx0, idx1, buf0, buf1, buf2, buf3,
            table_sp, sem_i0, sem_i1, sem_g0, sem_g1, sem_g2, sem_g3,
            sem_o0, sem_o1, sem_o2, sem_o3):
        idxs = [idx0, idx1]
        sem_i = [sem_i0, sem_i1]
        bufs = [buf0, buf1, buf2, buf3]
        sem_g = [sem_g0, sem_g1, sem_g2, sem_g3]
        sem_o = [sem_o0, sem_o1, sem_o2, sem_o3]

        sid = lax.axis_index("s")
        wid = sid * _NC + lax.axis_index("c")
        idx_row_base = wid * _IDX_ROWS_PER_W
        out_base = wid * _BPW

        @pl.when(sid == 0)
        def _():
            pltpu.sync_copy(table_hbm, table_sp)

        plsc.subcore_barrier()

        def fire_idx(g, i):
            pltpu.async_copy(
                idx_hbm.at[pl.ds(idx_row_base + g * _K, _K)], idxs[i],
                sem_i[i])

        def wait_idx(i):
            pltpu.make_async_copy(
                idx_hbm.at[pl.ds(idx_row_base, _K)], idxs[i], sem_i[i]
            ).wait()

        def fire_gather(s, b, i):
            return pltpu.async_copy(
                table_sp.at[idxs[i].at[s]], bufs[b], sem_g[b])

        def drain_gather(b):
            pltpu.make_async_copy(
                out_hbm.at[pl.ds(out_base, _SUB)], bufs[b], sem_g[b]
            ).wait()

        def fire_scatter(t, b):
            pltpu.async_copy(
                bufs[b], out_hbm.at[pl.ds(out_base + t * _SUB, _SUB)],
                sem_o[b])

        def drain_scatter(b):
            pltpu.make_async_copy(
                bufs[b], out_hbm.at[pl.ds(out_base, _SUB)], sem_o[b]
            ).wait()

        def one_group(g, i, first=False):
            wait_idx(i)
            fire_idx(jnp.minimum(g + 1, _NGROUP - 1), 1 - i)
            ds_ = [None] * _K
            for s in range(_K):
                b = s % _NBUF
                if not (first and s < _NBUF):
                    drain_scatter(b)
                ds_[s] = fire_gather(s, b, i)
                if s == 0:
                    if not first:
                        drain_gather(_NBUF - 1)
                        fire_scatter(g * _K - 1, _NBUF - 1)
                else:
                    ds_[s - 1].wait()
                    fire_scatter(g * _K + s - 1, (s - 1) % _NBUF)

        fire_idx(0, 0)
        one_group(0, 0, first=True)

        def body(k, carry):
            g = 1 + 2 * k
            one_group(g, 1)
            one_group(g + 1, 0)
            return carry

        lax.fori_loop(0, (_NGROUP - 2) // 2, body, 0)

        one_group(_NGROUP - 1, 1)
        drain_gather(_NBUF - 1)
        fire_scatter(_IDX_ROWS_PER_W - 1, _NBUF - 1)
        wait_idx(0)
        for b in range(_NBUF):
            drain_scatter(b)

    return emb


_emb_kernel = _make_sc_kernel()


@jax.jit
def kernel(atom_type_array, embedding_table):
    idx2d = atom_type_array.astype(jnp.int32).reshape(_B // 128, 128)
    out = _emb_kernel(embedding_table, idx2d)
    return out.reshape(atom_type_array.shape + (EMBED_DIM,))
